# baseline (speedup 1.0000x reference)
import jax
import jax.numpy as jnp
from jax.experimental import pallas as pl
from jax.experimental.pallas import tpu as pltpu

N = 10000
H = 128
BM = 400
BL = 1000

_CP = pltpu.CompilerParams(vmem_limit_bytes=100 * 1024 * 1024)


def _relu_norm(z):
    z = jnp.maximum(z, 0.0)
    n = jnp.sqrt(jnp.sum(z * z, axis=1, keepdims=True))
    return z / jnp.maximum(n, 1e-12)


def _center_emit(i, y, c_scr, yb_ref, c_ref):
    @pl.when(i == 0)
    def _():
        c_scr[...] = y[0:1, :]
    c = c_scr[...]
    yb_ref[...] = (y - c).astype(jnp.bfloat16)
    c_ref[...] = c


def _first_body(adj_ref, y_ref, b_ref, wn_ref, out_ref):
    z = jnp.dot(adj_ref[...], y_ref[...], preferred_element_type=jnp.float32)
    x = _relu_norm(z + b_ref[...])
    out_ref[...] = jnp.dot(x, wn_ref[...], preferred_element_type=jnp.float32)


def _cast_body(adj_ref, y_ref, b_ref, wn_ref,
               adjb_ref, s_ref, yb_ref, c_ref, c_scr):
    adj = adj_ref[...]
    adjb = adj.astype(jnp.bfloat16)
    adjb_ref[...] = adjb
    s_ref[...] = jnp.sum(adjb.astype(jnp.float32), axis=1, keepdims=True)
    z = jnp.dot(adj, y_ref[...], preferred_element_type=jnp.float32)
    x = _relu_norm(z + b_ref[...])
    y = jnp.dot(x, wn_ref[...], preferred_element_type=jnp.float32)
    _center_emit(pl.program_id(0), y, c_scr, yb_ref, c_ref)


def _mid_body(adjb_ref, yb_ref, c_in_ref, s_ref, b_ref, wn_ref,
              ob_ref, oc_ref, c_scr):
    z = jnp.dot(adjb_ref[...], yb_ref[...], preferred_element_type=jnp.float32)
    z = z + s_ref[...] * c_in_ref[...]
    x = _relu_norm(z + b_ref[...])
    y = jnp.dot(x, wn_ref[...], preferred_element_type=jnp.float32)
    _center_emit(pl.program_id(0), y, c_scr, ob_ref, oc_ref)


def _keep_body(adjb_ref, yb_ref, c_in_ref, s_ref, b_ref, wn_ref,
               x_ref, ob_ref, oc_ref, c_scr):
    z = jnp.dot(adjb_ref[...], yb_ref[...], preferred_element_type=jnp.float32)
    z = z + s_ref[...] * c_in_ref[...]
    x = _relu_norm(z + b_ref[...])
    x_ref[...] = x
    y = jnp.dot(x, wn_ref[...], preferred_element_type=jnp.float32)
    _center_emit(pl.program_id(0), y, c_scr, ob_ref, oc_ref)


def _score_body(adjb_ref, yb_ref, c_in_ref, s_ref, x_ref, bc_ref,
                wm1_ref, bm1_ref, wm2_ref, bm2_ref, out_ref):
    z = jnp.dot(adjb_ref[...], yb_ref[...], preferred_element_type=jnp.float32)
    z = z + s_ref[...] * c_in_ref[...] + bc_ref[...]
    wm1 = wm1_ref[...]
    bm1 = bm1_ref[...]
    acc = jnp.maximum(
        jnp.dot(x_ref[...], wm1, preferred_element_type=jnp.float32) + bm1, 0.0)
    for k in range(4):
        h = _relu_norm(z[:, k * H:(k + 1) * H])
        acc = acc + jnp.maximum(
            jnp.dot(h, wm1, preferred_element_type=jnp.float32) + bm1, 0.0)
    hl = jnp.maximum(z[:, 4 * H:], 0.0)
    acc = acc + jnp.maximum(
        jnp.dot(hl, wm1, preferred_element_type=jnp.float32) + bm1, 0.0)
    out_ref[...] = (jnp.dot(acc, wm2_ref[...], preferred_element_type=jnp.float32)
                    + 6.0 * bm2_ref[...])


def _row_blk(bm, w):
    return pl.BlockSpec((bm, w), lambda i: (i, 0))


def _full(*shape):
    return pl.BlockSpec(shape, lambda i: (0, 0))


def kernel(adj1, adj2, W0, b0, Ws, bs, Wl, bl, Wm1, bm1, Wm2, bm2):
    f32, bf16 = jnp.float32, jnp.bfloat16
    c_scr = pltpu.VMEM((1, H), f32)

    y0 = pl.pallas_call(
        _first_body,
        grid=(N // BM,),
        in_specs=[_row_blk(BM, N), _full(N, H), _full(1, H), _full(H, H)],
        out_specs=_row_blk(BM, H),
        out_shape=jax.ShapeDtypeStruct((N, H), f32),
        compiler_params=_CP,
    )(adj1, W0, b0.reshape(1, H), Ws[0])

    adj2b, s, yb, c = pl.pallas_call(
        _cast_body,
        grid=(N // BM,),
        in_specs=[_row_blk(BM, N), _full(N, H), _full(1, H), _full(H, H)],
        out_specs=[_row_blk(BM, N), _row_blk(BM, 1), _row_blk(BM, H),
                   _full(1, H)],
        out_shape=[jax.ShapeDtypeStruct((N, N), bf16),
                   jax.ShapeDtypeStruct((N, 1), f32),
                   jax.ShapeDtypeStruct((N, H), bf16),
                   jax.ShapeDtypeStruct((1, H), f32)],
        scratch_shapes=[c_scr],
        compiler_params=_CP,
    )(adj2, y0, bs[0].reshape(1, H), Ws[1])

    for b, wn in ((bs[1], Ws[2]), (bs[2], Ws[3])):
        yb, c = pl.pallas_call(
            _mid_body,
            grid=(N // BL,),
            in_specs=[_row_blk(BL, N), _full(N, H), _full(1, H),
                      _row_blk(BL, 1), _full(1, H), _full(H, H)],
            out_specs=[_row_blk(BL, H), _full(1, H)],
            out_shape=[jax.ShapeDtypeStruct((N, H), bf16),
                       jax.ShapeDtypeStruct((1, H), f32)],
            scratch_shapes=[c_scr],
            compiler_params=_CP,
        )(adj2b, yb, c, s, b.reshape(1, H), wn)

    Wcat = jnp.concatenate([Ws[0], Ws[1], Ws[2], Ws[3], Wl], axis=1)
    x4, ycb, yc = pl.pallas_call(
        _keep_body,
        grid=(N // BL,),
        in_specs=[_row_blk(BL, N), _full(N, H), _full(1, H),
                  _row_blk(BL, 1), _full(1, H), _full(H, 5 * H)],
        out_specs=[_row_blk(BL, H), _row_blk(BL, 5 * H), _full(1, 5 * H)],
        out_shape=[jax.ShapeDtypeStruct((N, H), f32),
                   jax.ShapeDtypeStruct((N, 5 * H), bf16),
                   jax.ShapeDtypeStruct((1, 5 * H), f32)],
        scratch_shapes=[pltpu.VMEM((1, 5 * H), f32)],
        compiler_params=_CP,
    )(adj2b, yb, c, s, bs[3].reshape(1, H), Wcat)

    bcat = jnp.concatenate([bs[0], bs[1], bs[2], bs[3], bl]).reshape(1, 5 * H)
    score = pl.pallas_call(
        _score_body,
        grid=(N // BL,),
        in_specs=[
            _row_blk(BL, N), _full(N, 5 * H), _full(1, 5 * H),
            _row_blk(BL, 1), _row_blk(BL, H), _full(1, 5 * H),
            _full(H, H), _full(1, H), _full(H, 1), _full(1, 1),
        ],
        out_specs=_row_blk(BL, 1),
        out_shape=jax.ShapeDtypeStruct((N, 1), f32),
        compiler_params=_CP,
    )(adj2b, ycb, yc, s, x4, bcat, Wm1, bm1.reshape(1, H), Wm2,
      bm2.reshape(1, 1))
    return score
